# Initial kernel scaffold; baseline (speedup 1.0000x reference)
import functools

import jax
import jax.numpy as jnp
from jax import lax
from jax.experimental import pallas as pl
from jax.experimental.pallas import tpu as pltpu
from jax.experimental.pallas import tpu_sc as plsc

N = 10000
DEG = 32
FEAT = 128
EMB = 128
NREL = 4
LANES = 16
NGRP = FEAT // LANES

NWORKERS = 32
CHUNK = 320
BP = NWORKERS * CHUNK
IDX_CHUNK = 80


def _sc_body(nodes_hbm, lf_hbm, n0, n1, n2, n3,
             selff, sums0, sums1, sums2, sums3,
             nodes_v, nbr_v, rows_v, acc_v, sem):
    wid = lax.axis_index("s") * 2 + lax.axis_index("c")
    base = wid * CHUNK
    pltpu.sync_copy(nodes_hbm.at[pl.ds(base, CHUNK)], nodes_v)

    for k in range(CHUNK // IDX_CHUNK):
        pltpu.async_copy(
            lf_hbm.at[nodes_v.at[pl.ds(k * IDX_CHUNK, IDX_CHUNK)]],
            acc_v.at[pl.ds(k * IDX_CHUNK, IDX_CHUNK)], sem).wait()
    pltpu.sync_copy(acc_v, selff.at[pl.ds(base, CHUNK)])

    for nr, out in ((n0, sums0), (n1, sums1), (n2, sums2), (n3, sums3)):
        for k in range(CHUNK // IDX_CHUNK):
            pltpu.async_copy(
                nr.at[nodes_v.at[pl.ds(k * IDX_CHUNK, IDX_CHUNK)]],
                nbr_v.at[pl.ds(k * IDX_CHUNK, IDX_CHUNK)], sem).wait()

        def body(b, carry):
            pltpu.async_copy(lf_hbm.at[nbr_v.at[b]], rows_v, sem).wait()
            for g in range(NGRP):
                s = rows_v[0, pl.ds(g * LANES, LANES)]
                for j in range(1, DEG):
                    s = s + rows_v[j, pl.ds(g * LANES, LANES)]
                acc_v[b, pl.ds(g * LANES, LANES)] = s
            return carry

        lax.fori_loop(0, CHUNK, body, 0)
        pltpu.sync_copy(acc_v, out.at[pl.ds(base, CHUNK)])


_ROW = jax.ShapeDtypeStruct((BP, FEAT), jnp.float32)

_sc_gather = functools.partial(
    pl.kernel,
    mesh=plsc.VectorSubcoreMesh(core_axis_name="c", subcore_axis_name="s"),
    out_type=[_ROW, _ROW, _ROW, _ROW, _ROW],
    scratch_types=[
        pltpu.VMEM((CHUNK,), jnp.int32),
        pltpu.VMEM((CHUNK, DEG), jnp.int32),
        pltpu.VMEM((DEG, FEAT), jnp.float32),
        pltpu.VMEM((CHUNK, FEAT), jnp.float32),
        pltpu.SemaphoreType.DMA,
    ],
)(_sc_body)


BLK = 1024


def _tc_body(self_ref, s0, s1, s2, s3,
             wa0, wa1, wa2, wa3,
             w1s, w10, w11, w12, w13, b1, w2, b2, out_ref):
    h = jnp.dot(self_ref[...], w1s[...], preferred_element_type=jnp.float32)
    scale = jnp.float32(1.0 / DEG)
    for s_ref, wa_ref, w1_ref in ((s0, wa0, w10), (s1, wa1, w11),
                                  (s2, wa2, w12), (s3, wa3, w13)):
        m = jnp.dot(s_ref[...] * scale, wa_ref[...],
                    preferred_element_type=jnp.float32)
        m = jnp.maximum(m, 0.0)
        h = h + jnp.dot(m, w1_ref[...], preferred_element_type=jnp.float32)
    h = jnp.tanh(h + b1[...])
    out_ref[...] = jnp.dot(h, w2[...],
                           preferred_element_type=jnp.float32) + b2[...]


def _row_spec():
    return pl.BlockSpec((BLK, FEAT), lambda i: (i, 0))


def _full_spec(shape):
    return pl.BlockSpec(shape, lambda i: (0, 0))


_tc_call = pl.pallas_call(
    _tc_body,
    grid=(BP // BLK,),
    in_specs=[_row_spec() for _ in range(5)]
    + [_full_spec((FEAT, EMB)) for _ in range(4)]
    + [_full_spec((FEAT, FEAT))]
    + [_full_spec((EMB, FEAT)) for _ in range(4)]
    + [_full_spec((1, FEAT))]
    + [_full_spec((FEAT, EMB))]
    + [_full_spec((1, EMB))],
    out_specs=pl.BlockSpec((BLK, EMB), lambda i: (i, 0)),
    out_shape=jax.ShapeDtypeStruct((BP, EMB), jnp.float32),
)


def kernel(nodes, local_features, neigh0, neigh1, neigh2, neigh3,
           Wa0, Wa1, Wa2, Wa3, W1, b1, W2, b2):
    nodes_p = jnp.pad(nodes.astype(jnp.int32), (0, BP - N))
    selff, s0, s1, s2, s3 = _sc_gather(
        nodes_p, local_features,
        neigh0.astype(jnp.int32), neigh1.astype(jnp.int32),
        neigh2.astype(jnp.int32), neigh3.astype(jnp.int32))
    out = _tc_call(
        selff, s0, s1, s2, s3,
        Wa0, Wa1, Wa2, Wa3,
        W1[:FEAT], W1[FEAT:FEAT + EMB], W1[FEAT + EMB:FEAT + 2 * EMB],
        W1[FEAT + 2 * EMB:FEAT + 3 * EMB], W1[FEAT + 3 * EMB:],
        b1.reshape(1, FEAT), W2, b2.reshape(1, EMB))
    return out[:N]


# speedup vs baseline: 2.4667x; 2.4667x over previous
import functools

import jax
import jax.numpy as jnp
from jax import lax
from jax.experimental import pallas as pl
from jax.experimental.pallas import tpu as pltpu
from jax.experimental.pallas import tpu_sc as plsc

N = 10000
DEG = 32
FEAT = 128
EMB = 128
NREL = 4
LANES = 16
NGRP = FEAT // LANES

NWORKERS = 32
CHUNK = 320
BP = NWORKERS * CHUNK
IDX_CHUNK = 80


def _sc_body(nodes_hbm, lf_hbm, nall,
             selff, sums0, sums1, sums2, sums3,
             nodes_v, nbr_v, rows_v, acc_v, sem):
    wid = lax.axis_index("s") * 2 + lax.axis_index("c")
    base = wid * CHUNK
    pltpu.sync_copy(nodes_hbm.at[pl.ds(base, CHUNK)], nodes_v)

    for k in range(CHUNK // IDX_CHUNK):
        pltpu.async_copy(
            lf_hbm.at[nodes_v.at[pl.ds(k * IDX_CHUNK, IDX_CHUNK)]],
            acc_v.at[pl.ds(k * IDX_CHUNK, IDX_CHUNK)], sem).wait()
    pltpu.sync_copy(acc_v, selff.at[pl.ds(base, CHUNK)])

    for k in range(CHUNK // IDX_CHUNK):
        pltpu.async_copy(
            nall.at[nodes_v.at[pl.ds(k * IDX_CHUNK, IDX_CHUNK)]],
            nbr_v.at[pl.ds(k * IDX_CHUNK, IDX_CHUNK)], sem).wait()

    for r, out in enumerate((sums0, sums1, sums2, sums3)):

        def body(b, carry):
            pltpu.async_copy(
                lf_hbm.at[nbr_v.at[b, pl.ds(r * DEG, DEG)]], rows_v,
                sem).wait()
            for g in range(NGRP):
                s = rows_v[0, pl.ds(g * LANES, LANES)]
                for j in range(1, DEG):
                    s = s + rows_v[j, pl.ds(g * LANES, LANES)]
                acc_v[b, pl.ds(g * LANES, LANES)] = s
            return carry

        lax.fori_loop(0, CHUNK, body, 0)
        pltpu.sync_copy(acc_v, out.at[pl.ds(base, CHUNK)])


_ROW = jax.ShapeDtypeStruct((BP, FEAT), jnp.float32)

_sc_gather = functools.partial(
    pl.kernel,
    mesh=plsc.VectorSubcoreMesh(core_axis_name="c", subcore_axis_name="s"),
    out_type=[_ROW, _ROW, _ROW, _ROW, _ROW],
    scratch_types=[
        pltpu.VMEM((CHUNK,), jnp.int32),
        pltpu.VMEM((CHUNK, NREL * DEG), jnp.int32),
        pltpu.VMEM((DEG, FEAT), jnp.float32),
        pltpu.VMEM((CHUNK, FEAT), jnp.float32),
        pltpu.SemaphoreType.DMA,
    ],
)(_sc_body)


BLK = 1024


def _tc_body(self_ref, s0, s1, s2, s3,
             wa0, wa1, wa2, wa3,
             w1s, w10, w11, w12, w13, b1, w2, b2, out_ref):
    h = jnp.dot(self_ref[...], w1s[...], preferred_element_type=jnp.float32)
    scale = jnp.float32(1.0 / DEG)
    for s_ref, wa_ref, w1_ref in ((s0, wa0, w10), (s1, wa1, w11),
                                  (s2, wa2, w12), (s3, wa3, w13)):
        m = jnp.dot(s_ref[...] * scale, wa_ref[...],
                    preferred_element_type=jnp.float32)
        m = jnp.maximum(m, 0.0)
        h = h + jnp.dot(m, w1_ref[...], preferred_element_type=jnp.float32)
    h = jnp.tanh(h + b1[...])
    out_ref[...] = jnp.dot(h, w2[...],
                           preferred_element_type=jnp.float32) + b2[...]


def _row_spec():
    return pl.BlockSpec((BLK, FEAT), lambda i: (i, 0))


def _full_spec(shape):
    return pl.BlockSpec(shape, lambda i: (0, 0))


_tc_call = pl.pallas_call(
    _tc_body,
    grid=(BP // BLK,),
    in_specs=[_row_spec() for _ in range(5)]
    + [_full_spec((FEAT, EMB)) for _ in range(4)]
    + [_full_spec((FEAT, FEAT))]
    + [_full_spec((EMB, FEAT)) for _ in range(4)]
    + [_full_spec((1, FEAT))]
    + [_full_spec((FEAT, EMB))]
    + [_full_spec((1, EMB))],
    out_specs=pl.BlockSpec((BLK, EMB), lambda i: (i, 0)),
    out_shape=jax.ShapeDtypeStruct((BP, EMB), jnp.float32),
)


def kernel(nodes, local_features, neigh0, neigh1, neigh2, neigh3,
           Wa0, Wa1, Wa2, Wa3, W1, b1, W2, b2):
    nodes_p = jnp.pad(nodes.astype(jnp.int32), (0, BP - N))
    nall = jnp.concatenate(
        [neigh0.astype(jnp.int32), neigh1.astype(jnp.int32),
         neigh2.astype(jnp.int32), neigh3.astype(jnp.int32)], axis=1)
    selff, s0, s1, s2, s3 = _sc_gather(nodes_p, local_features, nall)
    out = _tc_call(
        selff, s0, s1, s2, s3,
        Wa0, Wa1, Wa2, Wa3,
        W1[:FEAT], W1[FEAT:FEAT + EMB], W1[FEAT + EMB:FEAT + 2 * EMB],
        W1[FEAT + 2 * EMB:FEAT + 3 * EMB], W1[FEAT + 3 * EMB:],
        b1.reshape(1, FEAT), W2, b2.reshape(1, EMB))
    return out[:N]


# speedup vs baseline: 4.5421x; 1.8413x over previous
import functools

import jax
import jax.numpy as jnp
from jax import lax
from jax.experimental import pallas as pl
from jax.experimental.pallas import tpu as pltpu
from jax.experimental.pallas import tpu_sc as plsc

N = 10000
DEG = 32
FEAT = 128
EMB = 128
NREL = 4
LANES = 16
NGRP = FEAT // LANES

NWORKERS = 32
CHUNK = 320
BP = NWORKERS * CHUNK
IDX_CHUNK = 80


def _reduce_rows(rows, acc_v, b):
    for g in range(NGRP):
        s = rows[0, pl.ds(g * LANES, LANES)]
        for j in range(1, DEG):
            s = s + rows[j, pl.ds(g * LANES, LANES)]
        acc_v[b, pl.ds(g * LANES, LANES)] = s


def _sc_body(nodes_hbm, lf_hbm, nall,
             selff, sums0, sums1, sums2, sums3,
             nodes_v, nbr_v, rows0_v, rows1_v, acc_v,
             sem0, sem1, semn):
    wid = lax.axis_index("s") * 2 + lax.axis_index("c")
    base = wid * CHUNK
    pltpu.sync_copy(nodes_hbm.at[pl.ds(base, CHUNK)], nodes_v)

    for k in range(CHUNK // IDX_CHUNK):
        pltpu.async_copy(
            lf_hbm.at[nodes_v.at[pl.ds(k * IDX_CHUNK, IDX_CHUNK)]],
            acc_v.at[pl.ds(k * IDX_CHUNK, IDX_CHUNK)], sem0)
        pltpu.async_copy(
            nall.at[nodes_v.at[pl.ds(k * IDX_CHUNK, IDX_CHUNK)]],
            nbr_v.at[pl.ds(k * IDX_CHUNK, IDX_CHUNK)], semn)
    for k in range(CHUNK // IDX_CHUNK):
        pltpu.make_async_copy(
            lf_hbm.at[nodes_v.at[pl.ds(k * IDX_CHUNK, IDX_CHUNK)]],
            acc_v.at[pl.ds(k * IDX_CHUNK, IDX_CHUNK)], sem0).wait()
        pltpu.make_async_copy(
            nall.at[nodes_v.at[pl.ds(k * IDX_CHUNK, IDX_CHUNK)]],
            nbr_v.at[pl.ds(k * IDX_CHUNK, IDX_CHUNK)], semn).wait()
    pltpu.sync_copy(acc_v, selff.at[pl.ds(base, CHUNK)])

    for r, out in enumerate((sums0, sums1, sums2, sums3)):
        def idx(b):
            return nbr_v.at[b, pl.ds(r * DEG, DEG)]

        pltpu.async_copy(lf_hbm.at[idx(0)], rows0_v, sem0)

        def body(i, carry):
            b = i * 2
            pltpu.async_copy(lf_hbm.at[idx(b + 1)], rows1_v, sem1)
            pltpu.make_async_copy(lf_hbm.at[idx(b)], rows0_v, sem0).wait()
            _reduce_rows(rows0_v, acc_v, b)
            b2 = jnp.minimum(b + 2, CHUNK - 1)
            pltpu.async_copy(lf_hbm.at[idx(b2)], rows0_v, sem0)
            pltpu.make_async_copy(lf_hbm.at[idx(b + 1)], rows1_v,
                                  sem1).wait()
            _reduce_rows(rows1_v, acc_v, b + 1)
            return carry

        lax.fori_loop(0, CHUNK // 2, body, 0)
        pltpu.make_async_copy(lf_hbm.at[idx(0)], rows0_v, sem0).wait()
        pltpu.sync_copy(acc_v, out.at[pl.ds(base, CHUNK)])


_ROW = jax.ShapeDtypeStruct((BP, FEAT), jnp.float32)

_sc_gather = functools.partial(
    pl.kernel,
    mesh=plsc.VectorSubcoreMesh(core_axis_name="c", subcore_axis_name="s"),
    out_type=[_ROW, _ROW, _ROW, _ROW, _ROW],
    scratch_types=[
        pltpu.VMEM((CHUNK,), jnp.int32),
        pltpu.VMEM((CHUNK, NREL * DEG), jnp.int32),
        pltpu.VMEM((DEG, FEAT), jnp.float32),
        pltpu.VMEM((DEG, FEAT), jnp.float32),
        pltpu.VMEM((CHUNK, FEAT), jnp.float32),
        pltpu.SemaphoreType.DMA,
        pltpu.SemaphoreType.DMA,
        pltpu.SemaphoreType.DMA,
    ],
)(_sc_body)


BLK = 1024


def _tc_body(self_ref, s0, s1, s2, s3,
             wa0, wa1, wa2, wa3,
             w1s, w10, w11, w12, w13, b1, w2, b2, out_ref):
    h = jnp.dot(self_ref[...], w1s[...], preferred_element_type=jnp.float32)
    scale = jnp.float32(1.0 / DEG)
    for s_ref, wa_ref, w1_ref in ((s0, wa0, w10), (s1, wa1, w11),
                                  (s2, wa2, w12), (s3, wa3, w13)):
        m = jnp.dot(s_ref[...] * scale, wa_ref[...],
                    preferred_element_type=jnp.float32)
        m = jnp.maximum(m, 0.0)
        h = h + jnp.dot(m, w1_ref[...], preferred_element_type=jnp.float32)
    h = jnp.tanh(h + b1[...])
    out_ref[...] = jnp.dot(h, w2[...],
                           preferred_element_type=jnp.float32) + b2[...]


def _row_spec():
    return pl.BlockSpec((BLK, FEAT), lambda i: (i, 0))


def _full_spec(shape):
    return pl.BlockSpec(shape, lambda i: (0, 0))


_tc_call = pl.pallas_call(
    _tc_body,
    grid=(BP // BLK,),
    in_specs=[_row_spec() for _ in range(5)]
    + [_full_spec((FEAT, EMB)) for _ in range(4)]
    + [_full_spec((FEAT, FEAT))]
    + [_full_spec((EMB, FEAT)) for _ in range(4)]
    + [_full_spec((1, FEAT))]
    + [_full_spec((FEAT, EMB))]
    + [_full_spec((1, EMB))],
    out_specs=pl.BlockSpec((BLK, EMB), lambda i: (i, 0)),
    out_shape=jax.ShapeDtypeStruct((BP, EMB), jnp.float32),
)


def kernel(nodes, local_features, neigh0, neigh1, neigh2, neigh3,
           Wa0, Wa1, Wa2, Wa3, W1, b1, W2, b2):
    nodes_p = jnp.pad(nodes.astype(jnp.int32), (0, BP - N))
    nall = jnp.concatenate(
        [neigh0.astype(jnp.int32), neigh1.astype(jnp.int32),
         neigh2.astype(jnp.int32), neigh3.astype(jnp.int32)], axis=1)
    selff, s0, s1, s2, s3 = _sc_gather(nodes_p, local_features, nall)
    out = _tc_call(
        selff, s0, s1, s2, s3,
        Wa0, Wa1, Wa2, Wa3,
        W1[:FEAT], W1[FEAT:FEAT + EMB], W1[FEAT + EMB:FEAT + 2 * EMB],
        W1[FEAT + 2 * EMB:FEAT + 3 * EMB], W1[FEAT + 3 * EMB:],
        b1.reshape(1, FEAT), W2, b2.reshape(1, EMB))
    return out[:N]


# speedup vs baseline: 5.1335x; 1.1302x over previous
import functools

import jax
import jax.numpy as jnp
from jax import lax
from jax.experimental import pallas as pl
from jax.experimental.pallas import tpu as pltpu
from jax.experimental.pallas import tpu_sc as plsc

N = 10000
DEG = 32
FEAT = 128
EMB = 128
NREL = 4
LANES = 16
NGRP = FEAT // LANES

NWORKERS = 32
CHUNK = 320
BP = NWORKERS * CHUNK
IDX_CHUNK = 80


def _reduce_rows(rows, acc_v, b):
    for g0 in range(0, NGRP, 2):
        s0 = rows[0, pl.ds(g0 * LANES, LANES)]
        s1 = rows[0, pl.ds((g0 + 1) * LANES, LANES)]
        for j in range(1, DEG):
            s0 = s0 + rows[j, pl.ds(g0 * LANES, LANES)]
            s1 = s1 + rows[j, pl.ds((g0 + 1) * LANES, LANES)]
        acc_v[b, pl.ds(g0 * LANES, LANES)] = s0
        acc_v[b, pl.ds((g0 + 1) * LANES, LANES)] = s1


def _sc_body(nodes_hbm, lf_hbm, nall,
             selff, sums0, sums1, sums2, sums3,
             nodes_v, nbr_v, rows0_v, rows1_v, acc_v,
             sem0, sem1, semn):
    wid = lax.axis_index("s") * 2 + lax.axis_index("c")
    base = wid * CHUNK
    pltpu.sync_copy(nodes_hbm.at[pl.ds(base, CHUNK)], nodes_v)

    for k in range(CHUNK // IDX_CHUNK):
        pltpu.async_copy(
            lf_hbm.at[nodes_v.at[pl.ds(k * IDX_CHUNK, IDX_CHUNK)]],
            acc_v.at[pl.ds(k * IDX_CHUNK, IDX_CHUNK)], sem0)
        pltpu.async_copy(
            nall.at[nodes_v.at[pl.ds(k * IDX_CHUNK, IDX_CHUNK)]],
            nbr_v.at[pl.ds(k * IDX_CHUNK, IDX_CHUNK)], semn)
    for k in range(CHUNK // IDX_CHUNK):
        pltpu.make_async_copy(
            lf_hbm.at[nodes_v.at[pl.ds(k * IDX_CHUNK, IDX_CHUNK)]],
            acc_v.at[pl.ds(k * IDX_CHUNK, IDX_CHUNK)], sem0).wait()
        pltpu.make_async_copy(
            nall.at[nodes_v.at[pl.ds(k * IDX_CHUNK, IDX_CHUNK)]],
            nbr_v.at[pl.ds(k * IDX_CHUNK, IDX_CHUNK)], semn).wait()
    pltpu.sync_copy(acc_v, selff.at[pl.ds(base, CHUNK)])

    for r, out in enumerate((sums0, sums1, sums2, sums3)):
        def idx(b):
            return nbr_v.at[b, pl.ds(r * DEG, DEG)]

        pltpu.async_copy(lf_hbm.at[idx(0)], rows0_v, sem0)

        def body(i, carry):
            b = i * 2
            pltpu.async_copy(lf_hbm.at[idx(b + 1)], rows1_v, sem1)
            pltpu.make_async_copy(lf_hbm.at[idx(b)], rows0_v, sem0).wait()
            _reduce_rows(rows0_v, acc_v, b)
            b2 = jnp.minimum(b + 2, CHUNK - 1)
            pltpu.async_copy(lf_hbm.at[idx(b2)], rows0_v, sem0)
            pltpu.make_async_copy(lf_hbm.at[idx(b + 1)], rows1_v,
                                  sem1).wait()
            _reduce_rows(rows1_v, acc_v, b + 1)
            return carry

        lax.fori_loop(0, CHUNK // 2, body, 0)
        pltpu.make_async_copy(lf_hbm.at[idx(0)], rows0_v, sem0).wait()
        pltpu.sync_copy(acc_v, out.at[pl.ds(base, CHUNK)])


_ROW = jax.ShapeDtypeStruct((BP, FEAT), jnp.float32)

_sc_gather = functools.partial(
    pl.kernel,
    mesh=plsc.VectorSubcoreMesh(core_axis_name="c", subcore_axis_name="s"),
    out_type=[_ROW, _ROW, _ROW, _ROW, _ROW],
    scratch_types=[
        pltpu.VMEM((CHUNK,), jnp.int32),
        pltpu.VMEM((CHUNK, NREL * DEG), jnp.int32),
        pltpu.VMEM((DEG, FEAT), jnp.float32),
        pltpu.VMEM((DEG, FEAT), jnp.float32),
        pltpu.VMEM((CHUNK, FEAT), jnp.float32),
        pltpu.SemaphoreType.DMA,
        pltpu.SemaphoreType.DMA,
        pltpu.SemaphoreType.DMA,
    ],
)(_sc_body)


BLK = 1024


def _tc_body(self_ref, s0, s1, s2, s3,
             wa0, wa1, wa2, wa3,
             w1s, w10, w11, w12, w13, b1, w2, b2, out_ref):
    h = jnp.dot(self_ref[...], w1s[...], preferred_element_type=jnp.float32)
    scale = jnp.float32(1.0 / DEG)
    for s_ref, wa_ref, w1_ref in ((s0, wa0, w10), (s1, wa1, w11),
                                  (s2, wa2, w12), (s3, wa3, w13)):
        m = jnp.dot(s_ref[...] * scale, wa_ref[...],
                    preferred_element_type=jnp.float32)
        m = jnp.maximum(m, 0.0)
        h = h + jnp.dot(m, w1_ref[...], preferred_element_type=jnp.float32)
    h = jnp.tanh(h + b1[...])
    out_ref[...] = jnp.dot(h, w2[...],
                           preferred_element_type=jnp.float32) + b2[...]


def _row_spec():
    return pl.BlockSpec((BLK, FEAT), lambda i: (i, 0))


def _full_spec(shape):
    return pl.BlockSpec(shape, lambda i: (0, 0))


_tc_call = pl.pallas_call(
    _tc_body,
    grid=(BP // BLK,),
    in_specs=[_row_spec() for _ in range(5)]
    + [_full_spec((FEAT, EMB)) for _ in range(4)]
    + [_full_spec((FEAT, FEAT))]
    + [_full_spec((EMB, FEAT)) for _ in range(4)]
    + [_full_spec((1, FEAT))]
    + [_full_spec((FEAT, EMB))]
    + [_full_spec((1, EMB))],
    out_specs=pl.BlockSpec((BLK, EMB), lambda i: (i, 0)),
    out_shape=jax.ShapeDtypeStruct((BP, EMB), jnp.float32),
)


def kernel(nodes, local_features, neigh0, neigh1, neigh2, neigh3,
           Wa0, Wa1, Wa2, Wa3, W1, b1, W2, b2):
    nodes_p = jnp.pad(nodes.astype(jnp.int32), (0, BP - N))
    nall = jnp.concatenate(
        [neigh0.astype(jnp.int32), neigh1.astype(jnp.int32),
         neigh2.astype(jnp.int32), neigh3.astype(jnp.int32)], axis=1)
    selff, s0, s1, s2, s3 = _sc_gather(nodes_p, local_features, nall)
    out = _tc_call(
        selff, s0, s1, s2, s3,
        Wa0, Wa1, Wa2, Wa3,
        W1[:FEAT], W1[FEAT:FEAT + EMB], W1[FEAT + EMB:FEAT + 2 * EMB],
        W1[FEAT + 2 * EMB:FEAT + 3 * EMB], W1[FEAT + 3 * EMB:],
        b1.reshape(1, FEAT), W2, b2.reshape(1, EMB))
    return out[:N]


# speedup vs baseline: 8.0107x; 1.5605x over previous
import functools

import jax
import jax.numpy as jnp
from jax import lax
from jax.experimental import pallas as pl
from jax.experimental.pallas import tpu as pltpu
from jax.experimental.pallas import tpu_sc as plsc

N = 10000
DEG = 32
FEAT = 128
EMB = 128
NREL = 4
LANES = 16
NGRP = FEAT // LANES

NWORKERS = 32
CHUNK = 320
BP = NWORKERS * CHUNK
IDX_CHUNK = 80


def _reduce_rows(rows, acc_v, b):
    for g0 in range(0, NGRP, 2):
        s0 = rows[0, pl.ds(g0 * LANES, LANES)]
        s1 = rows[0, pl.ds((g0 + 1) * LANES, LANES)]
        for j in range(1, DEG):
            s0 = s0 + rows[j, pl.ds(g0 * LANES, LANES)]
            s1 = s1 + rows[j, pl.ds((g0 + 1) * LANES, LANES)]
        acc_v[b, pl.ds(g0 * LANES, LANES)] = s0
        acc_v[b, pl.ds((g0 + 1) * LANES, LANES)] = s1


NBUF = 4


def _sc_body(nodes_hbm, lf_hbm, nall,
             selff, sums0, sums1, sums2, sums3,
             nodes_v, nbr_v, rows0_v, rows1_v, rows2_v, rows3_v, acc_v,
             sem0, sem1, sem2, sem3, semn):
    rows = (rows0_v, rows1_v, rows2_v, rows3_v)
    sems = (sem0, sem1, sem2, sem3)
    wid = lax.axis_index("s") * 2 + lax.axis_index("c")
    base = wid * CHUNK
    pltpu.sync_copy(nodes_hbm.at[pl.ds(base, CHUNK)], nodes_v)

    for k in range(CHUNK // IDX_CHUNK):
        pltpu.async_copy(
            lf_hbm.at[nodes_v.at[pl.ds(k * IDX_CHUNK, IDX_CHUNK)]],
            acc_v.at[pl.ds(k * IDX_CHUNK, IDX_CHUNK)], sem0)
        pltpu.async_copy(
            nall.at[nodes_v.at[pl.ds(k * IDX_CHUNK, IDX_CHUNK)]],
            nbr_v.at[pl.ds(k * IDX_CHUNK, IDX_CHUNK)], semn)
    for k in range(CHUNK // IDX_CHUNK):
        pltpu.make_async_copy(
            lf_hbm.at[nodes_v.at[pl.ds(k * IDX_CHUNK, IDX_CHUNK)]],
            acc_v.at[pl.ds(k * IDX_CHUNK, IDX_CHUNK)], sem0).wait()
        pltpu.make_async_copy(
            nall.at[nodes_v.at[pl.ds(k * IDX_CHUNK, IDX_CHUNK)]],
            nbr_v.at[pl.ds(k * IDX_CHUNK, IDX_CHUNK)], semn).wait()
    pltpu.sync_copy(acc_v, selff.at[pl.ds(base, CHUNK)])

    for r, out in enumerate((sums0, sums1, sums2, sums3)):
        def idx(b):
            return nbr_v.at[b, pl.ds(r * DEG, DEG)]

        for p in range(NBUF):
            pltpu.async_copy(lf_hbm.at[idx(p)], rows[p], sems[p])

        def body(i, carry):
            b = i * NBUF
            for p in range(NBUF):
                pltpu.make_async_copy(lf_hbm.at[idx(b + p)], rows[p],
                                      sems[p]).wait()
                _reduce_rows(rows[p], acc_v, b + p)
                nxt = jnp.minimum(b + p + NBUF, CHUNK - 1)
                pltpu.async_copy(lf_hbm.at[idx(nxt)], rows[p], sems[p])
            return carry

        lax.fori_loop(0, CHUNK // NBUF, body, 0)
        for p in range(NBUF):
            pltpu.make_async_copy(lf_hbm.at[idx(0)], rows[p],
                                  sems[p]).wait()
        pltpu.sync_copy(acc_v, out.at[pl.ds(base, CHUNK)])


_ROW = jax.ShapeDtypeStruct((BP, FEAT), jnp.float32)

_sc_gather = functools.partial(
    pl.kernel,
    mesh=plsc.VectorSubcoreMesh(core_axis_name="c", subcore_axis_name="s"),
    out_type=[_ROW, _ROW, _ROW, _ROW, _ROW],
    scratch_types=[
        pltpu.VMEM((CHUNK,), jnp.int32),
        pltpu.VMEM((CHUNK, NREL * DEG), jnp.int32),
        pltpu.VMEM((DEG, FEAT), jnp.float32),
        pltpu.VMEM((DEG, FEAT), jnp.float32),
        pltpu.VMEM((DEG, FEAT), jnp.float32),
        pltpu.VMEM((DEG, FEAT), jnp.float32),
        pltpu.VMEM((CHUNK, FEAT), jnp.float32),
        pltpu.SemaphoreType.DMA,
        pltpu.SemaphoreType.DMA,
        pltpu.SemaphoreType.DMA,
        pltpu.SemaphoreType.DMA,
        pltpu.SemaphoreType.DMA,
    ],
)(_sc_body)


BLK = 1024


def _tc_body(self_ref, s0, s1, s2, s3,
             wa0, wa1, wa2, wa3,
             w1s, w10, w11, w12, w13, b1, w2, b2, out_ref):
    h = jnp.dot(self_ref[...], w1s[...], preferred_element_type=jnp.float32)
    scale = jnp.float32(1.0 / DEG)
    for s_ref, wa_ref, w1_ref in ((s0, wa0, w10), (s1, wa1, w11),
                                  (s2, wa2, w12), (s3, wa3, w13)):
        m = jnp.dot(s_ref[...] * scale, wa_ref[...],
                    preferred_element_type=jnp.float32)
        m = jnp.maximum(m, 0.0)
        h = h + jnp.dot(m, w1_ref[...], preferred_element_type=jnp.float32)
    h = jnp.tanh(h + b1[...])
    out_ref[...] = jnp.dot(h, w2[...],
                           preferred_element_type=jnp.float32) + b2[...]


def _row_spec():
    return pl.BlockSpec((BLK, FEAT), lambda i: (i, 0))


def _full_spec(shape):
    return pl.BlockSpec(shape, lambda i: (0, 0))


_tc_call = pl.pallas_call(
    _tc_body,
    grid=(BP // BLK,),
    in_specs=[_row_spec() for _ in range(5)]
    + [_full_spec((FEAT, EMB)) for _ in range(4)]
    + [_full_spec((FEAT, FEAT))]
    + [_full_spec((EMB, FEAT)) for _ in range(4)]
    + [_full_spec((1, FEAT))]
    + [_full_spec((FEAT, EMB))]
    + [_full_spec((1, EMB))],
    out_specs=pl.BlockSpec((BLK, EMB), lambda i: (i, 0)),
    out_shape=jax.ShapeDtypeStruct((BP, EMB), jnp.float32),
)


def kernel(nodes, local_features, neigh0, neigh1, neigh2, neigh3,
           Wa0, Wa1, Wa2, Wa3, W1, b1, W2, b2):
    nodes_p = jnp.pad(nodes.astype(jnp.int32), (0, BP - N))
    nall = jnp.concatenate(
        [neigh0.astype(jnp.int32), neigh1.astype(jnp.int32),
         neigh2.astype(jnp.int32), neigh3.astype(jnp.int32)], axis=1)
    selff, s0, s1, s2, s3 = _sc_gather(nodes_p, local_features, nall)
    out = _tc_call(
        selff, s0, s1, s2, s3,
        Wa0, Wa1, Wa2, Wa3,
        W1[:FEAT], W1[FEAT:FEAT + EMB], W1[FEAT + EMB:FEAT + 2 * EMB],
        W1[FEAT + 2 * EMB:FEAT + 3 * EMB], W1[FEAT + 3 * EMB:],
        b1.reshape(1, FEAT), W2, b2.reshape(1, EMB))
    return out[:N]
